# baseline (speedup 1.0000x reference)
import functools

import jax
import jax.numpy as jnp
from jax import lax
from jax.experimental import pallas as pl
from jax.experimental.pallas import tpu as pltpu
from jax.experimental.pallas import tpu_sc as plsc

N = 10000
E = 320000
D = 128
NC, NS = 2, 16
CH = 128
CPT = 80
EPT = CH * CPT
EP = EPT * NC * NS
NCH = EP // CH
NP = 10240
RPT = NP // NS
NJ = NP - N
RING = 6

_mesh = plsc.VectorSubcoreMesh(
    core_axis_name="c", subcore_axis_name="s", num_cores=NC, num_subcores=NS
)


def _deg_counts(dst2d, zeros16, ones16):
    @functools.partial(
        pl.kernel,
        out_type=jax.ShapeDtypeStruct((NC, NP, 16), jnp.float32),
        mesh=_mesh,
        scratch_types=[
            pltpu.VMEM((CPT, CH), jnp.int32),
            pltpu.VMEM((CH, 16), jnp.float32),
            pltpu.VMEM_SHARED((NP, 16), jnp.float32),
        ],
    )
    def k(dst_hbm, z_hbm, ones_hbm, out_hbm, didx, ones_v, acc):
        c = lax.axis_index("c")
        s = lax.axis_index("s")
        wid = c * NS + s
        pltpu.sync_copy(dst_hbm.at[pl.ds(wid * CPT, CPT)], didx)
        pltpu.sync_copy(ones_hbm, ones_v)
        pltpu.sync_copy(z_hbm, acc.at[pl.ds(s * RPT, RPT)])
        plsc.subcore_barrier()

        @pl.loop(0, CPT)
        def _(j):
            pltpu.sync_copy(ones_v, acc.at[didx.at[j]], add=True)

        plsc.subcore_barrier()
        pltpu.sync_copy(
            acc.at[pl.ds(s * RPT, RPT)], out_hbm.at[c, pl.ds(s * RPT, RPT)]
        )

    return k(dst2d, zeros16, ones16)


NBUF = 2


def _edge_scatter(g, idx3d, zeros128):
    @functools.partial(
        pl.kernel,
        out_type=jax.ShapeDtypeStruct((NC, NP, D), jnp.float32),
        mesh=_mesh,
        scratch_types=[pltpu.VMEM((2 * RING, CH), jnp.int32)]
        + [pltpu.VMEM((CH, D), jnp.float32) for _ in range(NBUF)]
        + [pltpu.VMEM_SHARED((NP, D), jnp.float32)]
        + [pltpu.SemaphoreType.DMA for _ in range(RING + 2 * NBUF)],
    )
    def k(g_hbm, idx_hbm, z_hbm, out_hbm, iring, *rest):
        rows = rest[:NBUF]
        acc = rest[NBUF]
        isem = rest[NBUF + 1 : NBUF + 1 + RING]
        gsem = rest[NBUF + 1 + RING : NBUF + 1 + RING + NBUF]
        ssem = rest[NBUF + 1 + RING + NBUF :]
        c = lax.axis_index("c")
        s = lax.axis_index("s")
        wid = c * NS + s
        base = wid * CPT
        pltpu.sync_copy(z_hbm, acc.at[pl.ds(s * RPT, RPT)])
        plsc.subcore_barrier()

        nouter = (CPT + 2 + RING - 1) // RING

        @pl.loop(0, nouter)
        def _(i):
            for b in range(RING):
                t = i * RING + b
                rl = 2 * b
                rl_g = 2 * ((b - 1) % RING)
                rl_s = 2 * ((b - 2) % RING)
                sl_g = (b - 1) % NBUF
                sl_s = (b - 2) % NBUF

                @pl.when(t < CPT)
                def _():
                    pltpu.async_copy(
                        idx_hbm.at[base + t], iring.at[pl.ds(rl, 2)],
                        isem[b],
                    )

                cg = t - 1

                @pl.when((cg >= 0) & (cg < CPT))
                def _():
                    @pl.when(cg >= NBUF)
                    def _():
                        pltpu.make_async_copy(
                            rows[sl_g], acc.at[iring.at[rl_g + 1]],
                            ssem[sl_g],
                        ).wait()

                    pltpu.make_async_copy(
                        idx_hbm.at[base], iring.at[pl.ds(rl_g, 2)],
                        isem[(b - 1) % RING],
                    ).wait()
                    pltpu.async_copy(
                        g_hbm.at[iring.at[rl_g]], rows[sl_g], gsem[sl_g]
                    )

                cs = t - 2

                @pl.when((cs >= 0) & (cs < CPT))
                def _():
                    pltpu.make_async_copy(
                        g_hbm.at[iring.at[rl_s]], rows[sl_s], gsem[sl_s]
                    ).wait()
                    pltpu.async_copy(
                        rows[sl_s], acc.at[iring.at[rl_s + 1]], ssem[sl_s],
                        add=True,
                    )

        for b in range(NBUF):
            pltpu.make_async_copy(
                rows[b], acc.at[iring.at[1]], ssem[b]
            ).wait()

        plsc.subcore_barrier()
        pltpu.sync_copy(
            acc.at[pl.ds(s * RPT, RPT)], out_hbm.at[c, pl.ds(s * RPT, RPT)]
        )

    return k(g, idx3d, zeros128)


def _matmul(x, w):
    def body(x_ref, w_ref, o_ref):
        o_ref[...] = jnp.dot(
            x_ref[...], w_ref[...],
            precision=lax.Precision.HIGHEST,
            preferred_element_type=jnp.float32,
        )

    return pl.pallas_call(
        body, out_shape=jax.ShapeDtypeStruct((N, D), jnp.float32)
    )(x, w)


def _prep(counts, h):
    def body(c_ref, h_ref, g_ref, dinv_ref):
        deg = c_ref[0, :N, 0:1] + c_ref[1, :N, 0:1] + 1.0
        dinv = lax.rsqrt(deg)
        dinv_ref[...] = dinv
        g_ref[...] = h_ref[...] * dinv

    return pl.pallas_call(
        body,
        out_shape=(
            jax.ShapeDtypeStruct((N, D), jnp.float32),
            jax.ShapeDtypeStruct((N, 1), jnp.float32),
        ),
    )(counts, h)


def _mid(S, g, dinv, b, w):
    def body(S_ref, g_ref, dinv_ref, b_ref, w_ref, o_ref):
        sm = S_ref[0, :N, :] + S_ref[1, :N, :]
        x2 = dinv_ref[...] * (sm + g_ref[...]) + b_ref[...]
        o_ref[...] = dinv_ref[...] * jnp.dot(
            x2, w_ref[...],
            precision=lax.Precision.HIGHEST,
            preferred_element_type=jnp.float32,
        )

    return pl.pallas_call(
        body, out_shape=jax.ShapeDtypeStruct((N, D), jnp.float32)
    )(S, g, dinv, b, w)


def _fin(S, g, dinv, b):
    def body(S_ref, g_ref, dinv_ref, b_ref, o_ref):
        sm = S_ref[0, :N, :] + S_ref[1, :N, :]
        o_ref[...] = dinv_ref[...] * (sm + g_ref[...]) + b_ref[...]

    return pl.pallas_call(
        body, out_shape=jax.ShapeDtypeStruct((N, D), jnp.float32)
    )(S, g, dinv, b)


def kernel(x, edge_index, W1, b1, W2, b2, W3, b3):
    ei = edge_index.astype(jnp.int32)
    pad = EP - E
    pad_src = jnp.arange(pad, dtype=jnp.int32) % N
    pad_dst = N + (jnp.arange(pad, dtype=jnp.int32) % NJ)
    src2d = jnp.concatenate([ei[0], pad_src]).reshape(NCH, CH)
    dst2d = jnp.concatenate([ei[1], pad_dst]).reshape(NCH, CH)
    idx3d = jnp.stack([src2d, dst2d], axis=1)
    zeros128 = jnp.zeros((RPT, D), jnp.float32)
    zeros16 = jnp.zeros((RPT, 16), jnp.float32)
    ones16 = jnp.zeros((CH, 16), jnp.float32).at[:, 0].set(1.0)

    counts = _deg_counts(dst2d, zeros16, ones16)
    h1 = _matmul(x, W1)
    g1, dinv = _prep(counts, h1)
    S1 = _edge_scatter(g1, idx3d, zeros128)
    g2 = _mid(S1, g1, dinv, b1.reshape(1, D), W2)
    S2 = _edge_scatter(g2, idx3d, zeros128)
    g3 = _mid(S2, g2, dinv, b2.reshape(1, D), W3)
    S3 = _edge_scatter(g3, idx3d, zeros128)
    return _fin(S3, g3, dinv, b3.reshape(1, D))
